# baseline (speedup 1.0000x reference)
import jax
import jax.numpy as jnp
from jax.experimental import pallas as pl
from jax.experimental.pallas import tpu as pltpu

_N = 4096
_V = 32000
_LS = 0.1
_RB = 512
_CB = 3200


def _body(tgt_ref, col_ref, lp_ref, out_ref, acc_ref):
    i = pl.program_id(0)
    j = pl.program_id(1)

    @pl.when((i == 0) & (j == 0))
    def _init():
        acc_ref[0] = 0.0
        acc_ref[1] = 0.0
        acc_ref[2] = 0.0

    blk = lp_ref[...]
    tgt = tgt_ref[...]
    valid = (tgt != 1).astype(jnp.float32)

    rowsum = jnp.sum(blk, axis=1, keepdims=True)
    acc_ref[0] += jnp.sum(rowsum * valid)

    tgtc = jnp.maximum(tgt, 0)
    hit = col_ref[...] == tgtc
    s2row = jnp.sum(jnp.where(hit, blk, 0.0), axis=1, keepdims=True)
    acc_ref[1] += jnp.sum(s2row * valid)

    @pl.when(j == 0)
    def _count():
        acc_ref[2] += jnp.sum(valid)

    @pl.when((i == pl.num_programs(0) - 1) & (j == pl.num_programs(1) - 1))
    def _finalize():
        c1 = _LS / _V
        c2 = 1.0 - _LS - c1
        denom = jnp.maximum(acc_ref[2], 1.0)
        out_ref[0, 0] = -(c1 * acc_ref[0] + c2 * acc_ref[1]) / denom


def kernel(log_probs, targets, triplets):
    tgt2d = jnp.asarray(targets, jnp.int32).reshape(_N, 1)
    cols = jnp.arange(_V, dtype=jnp.int32).reshape(1, _V)
    out = pl.pallas_call(
        _body,
        grid=(_N // _RB, _V // _CB),
        in_specs=[
            pl.BlockSpec((_RB, 1), lambda i, j: (i, 0)),
            pl.BlockSpec((1, _CB), lambda i, j: (0, j)),
            pl.BlockSpec((_RB, _CB), lambda i, j: (i, j)),
        ],
        out_specs=pl.BlockSpec(memory_space=pltpu.SMEM),
        out_shape=jax.ShapeDtypeStruct((1, 1), jnp.float32),
        scratch_shapes=[pltpu.SMEM((3,), jnp.float32)],
    )(tgt2d, cols, log_probs)
    return out[0, 0]


# speedup vs baseline: 1.1593x; 1.1593x over previous
import jax
import jax.numpy as jnp
from jax.experimental import pallas as pl
from jax.experimental.pallas import tpu as pltpu

_N = 4096
_V = 32000
_LS = 0.1
_RB = 256
_CB = 16000


def _body(tgt_ref, col_ref, lp_ref, out_ref, acc_ref):
    i = pl.program_id(0)
    j = pl.program_id(1)

    @pl.when((i == 0) & (j == 0))
    def _init():
        acc_ref[0] = 0.0
        acc_ref[1] = 0.0
        acc_ref[2] = 0.0

    blk = lp_ref[...]
    tgt = tgt_ref[...]
    valid = (tgt != 1).astype(jnp.float32)

    rowsum = jnp.sum(blk, axis=1, keepdims=True)
    acc_ref[0] += jnp.sum(rowsum * valid)

    tgtc = jnp.maximum(tgt, 0)
    hit = col_ref[...] == tgtc
    s2row = jnp.sum(jnp.where(hit, blk, 0.0), axis=1, keepdims=True)
    acc_ref[1] += jnp.sum(s2row * valid)

    @pl.when(j == 0)
    def _count():
        acc_ref[2] += jnp.sum(valid)

    @pl.when((i == pl.num_programs(0) - 1) & (j == pl.num_programs(1) - 1))
    def _finalize():
        c1 = _LS / _V
        c2 = 1.0 - _LS - c1
        denom = jnp.maximum(acc_ref[2], 1.0)
        out_ref[0, 0] = -(c1 * acc_ref[0] + c2 * acc_ref[1]) / denom


def kernel(log_probs, targets, triplets):
    tgt2d = jnp.asarray(targets, jnp.int32).reshape(_N, 1)
    cols = jnp.arange(_V, dtype=jnp.int32).reshape(1, _V)
    out = pl.pallas_call(
        _body,
        grid=(_N // _RB, _V // _CB),
        in_specs=[
            pl.BlockSpec((_RB, 1), lambda i, j: (i, 0)),
            pl.BlockSpec((1, _CB), lambda i, j: (0, j)),
            pl.BlockSpec((_RB, _CB), lambda i, j: (i, j)),
        ],
        out_specs=pl.BlockSpec(memory_space=pltpu.SMEM),
        out_shape=jax.ShapeDtypeStruct((1, 1), jnp.float32),
        scratch_shapes=[pltpu.SMEM((3,), jnp.float32)],
    )(tgt2d, cols, log_probs)
    return out[0, 0]


# speedup vs baseline: 1.1622x; 1.0025x over previous
import jax
import jax.numpy as jnp
from jax.experimental import pallas as pl
from jax.experimental.pallas import tpu as pltpu

_N = 4096
_V = 32000
_LS = 0.1
_RB = 512
_CB = 6400


def _body(tgt_ref, col_ref, lp_ref, out_ref, acc_ref):
    i = pl.program_id(0)
    j = pl.program_id(1)

    @pl.when((i == 0) & (j == 0))
    def _init():
        acc_ref[0] = 0.0
        acc_ref[1] = 0.0
        acc_ref[2] = 0.0

    blk = lp_ref[...]
    tgt = tgt_ref[...]
    valid = (tgt != 1).astype(jnp.float32)

    rowsum = jnp.sum(blk, axis=1, keepdims=True)
    acc_ref[0] += jnp.sum(rowsum * valid)

    tgtc = jnp.maximum(tgt, 0)
    hit = col_ref[...] == tgtc
    s2row = jnp.sum(jnp.where(hit, blk, 0.0), axis=1, keepdims=True)
    acc_ref[1] += jnp.sum(s2row * valid)

    @pl.when(j == 0)
    def _count():
        acc_ref[2] += jnp.sum(valid)

    @pl.when((i == pl.num_programs(0) - 1) & (j == pl.num_programs(1) - 1))
    def _finalize():
        c1 = _LS / _V
        c2 = 1.0 - _LS - c1
        denom = jnp.maximum(acc_ref[2], 1.0)
        out_ref[0, 0] = -(c1 * acc_ref[0] + c2 * acc_ref[1]) / denom


def kernel(log_probs, targets, triplets):
    tgt2d = jnp.asarray(targets, jnp.int32).reshape(_N, 1)
    cols = jnp.arange(_V, dtype=jnp.int32).reshape(1, _V)
    out = pl.pallas_call(
        _body,
        grid=(_N // _RB, _V // _CB),
        in_specs=[
            pl.BlockSpec((_RB, 1), lambda i, j: (i, 0)),
            pl.BlockSpec((1, _CB), lambda i, j: (0, j)),
            pl.BlockSpec((_RB, _CB), lambda i, j: (i, j)),
        ],
        out_specs=pl.BlockSpec(memory_space=pltpu.SMEM),
        out_shape=jax.ShapeDtypeStruct((1, 1), jnp.float32),
        scratch_shapes=[pltpu.SMEM((3,), jnp.float32)],
    )(tgt2d, cols, log_probs)
    return out[0, 0]
